# baseline (speedup 1.0000x reference)
import jax
import jax.numpy as jnp
from jax import lax
from jax.experimental import pallas as pl
from jax.experimental.pallas import tpu as pltpu


def _round_up(x, m):
    return (x + m - 1) // m * m


def _make_kernel(tv, nv, mc):
    def _logits_kernel(x_ref, bias_ref, emb_hbm, out_hbm,
                       xb_ref, ebuf, obuf, in_sem, out_sem):
        v = pl.program_id(0)
        slot = lax.rem(v, 2)
        nxt = 1 - slot

        def ecopy(i, s):
            return pltpu.make_async_copy(
                emb_hbm.at[pl.ds(i * tv, tv), :], ebuf.at[s], in_sem.at[s])

        def ocopy(i, s):
            return pltpu.make_async_copy(
                obuf.at[s], out_hbm.at[:, pl.ds(i * tv, tv)], out_sem.at[s])

        @pl.when(v == 0)
        def _():
            ecopy(0, 0).start()
            xb_ref[...] = x_ref[...].astype(jnp.bfloat16)

        @pl.when(v + 1 < nv)
        def _():
            ecopy(v + 1, nxt).start()

        ecopy(v, slot).wait()

        @pl.when(v >= 2)
        def _():
            ocopy(v - 2, slot).wait()

        eb = ebuf[slot].astype(jnp.bfloat16)
        bias_t = bias_ref[:, pl.ds(v * tv, tv)]
        B_p = x_ref.shape[0]
        for i in range(0, B_p, mc):
            acc = lax.dot_general(
                xb_ref[i:i + mc, :], eb,
                dimension_numbers=(((1,), (1,)), ((), ())),
                preferred_element_type=jnp.float32)
            obuf[slot, i:i + mc, :] = acc + bias_t

        ocopy(v, slot).start()

        @pl.when(v == nv - 1)
        def _():
            ocopy(v, slot).wait()
        if nv >= 2:
            @pl.when(v == nv - 1)
            def _():
                ocopy(v - 1, nxt).wait()

    return _logits_kernel


def kernel(x, emb_weight, bias):
    B, D = x.shape
    V = emb_weight.shape[0]

    tv = next((t for t in (1280, 640, 512, 768, 384, 256, 128) if V % t == 0),
              512)
    V_pad = _round_up(V, tv)
    nv = V_pad // tv

    B_p = _round_up(B, 8)
    x_p = x if B_p == B else jnp.pad(x, ((0, B_p - B), (0, 0)))
    bias_p = bias.astype(jnp.float32)
    emb_p = emb_weight
    if V_pad != V:
        bias_p = jnp.pad(bias_p, ((0, 0), (0, V_pad - V)))
        emb_p = jnp.pad(emb_p, ((0, V_pad - V), (0, 0)))

    mc = min(512, B_p)
    out = pl.pallas_call(
        _make_kernel(tv, nv, mc),
        out_shape=jax.ShapeDtypeStruct((B_p, V_pad), jnp.float32),
        grid=(nv,),
        in_specs=[
            pl.BlockSpec((B_p, D), lambda v: (0, 0)),
            pl.BlockSpec((1, V_pad), lambda v: (0, 0)),
            pl.BlockSpec(memory_space=pl.ANY),
        ],
        out_specs=pl.BlockSpec(memory_space=pl.ANY),
        scratch_shapes=[
            pltpu.VMEM((B_p, D), jnp.bfloat16),
            pltpu.VMEM((2, tv, D), jnp.float32),
            pltpu.VMEM((2, B_p, tv), jnp.float32),
            pltpu.SemaphoreType.DMA((2,)),
            pltpu.SemaphoreType.DMA((2,)),
        ],
        compiler_params=pltpu.CompilerParams(
            dimension_semantics=("arbitrary",),
            vmem_limit_bytes=64 * 1024 * 1024,
        ),
    )(x_p, bias_p, emb_p)
    if B_p != B or V_pad != V:
        out = out[:B, :V]
    return out


# speedup vs baseline: 1.0053x; 1.0053x over previous
import jax
import jax.numpy as jnp
from jax import lax
from jax.experimental import pallas as pl
from jax.experimental.pallas import tpu as pltpu


def _round_up(x, m):
    return (x + m - 1) // m * m


def _make_kernel(tv, nv, mc):
    def _logits_kernel(x_ref, bias_ref, emb_hbm, out_hbm,
                       xb_ref, ebuf, obuf, in_sem, out_sem):
        v = pl.program_id(0)
        slot = lax.rem(v, 2)
        nxt = 1 - slot
        eslot = lax.rem(v, 3)

        def ecopy(i, s):
            return pltpu.make_async_copy(
                emb_hbm.at[pl.ds(i * tv, tv), :], ebuf.at[s], in_sem.at[s])

        def ocopy(i, s):
            return pltpu.make_async_copy(
                obuf.at[s], out_hbm.at[:, pl.ds(i * tv, tv)], out_sem.at[s])

        @pl.when(v == 0)
        def _():
            ecopy(0, 0).start()
            xb_ref[...] = x_ref[...].astype(jnp.bfloat16)
        if nv >= 2:
            @pl.when(v == 0)
            def _():
                ecopy(1, 1).start()

        @pl.when(v + 2 < nv)
        def _():
            ecopy(v + 2, lax.rem(v + 2, 3)).start()

        ecopy(v, eslot).wait()

        @pl.when(v >= 2)
        def _():
            ocopy(v - 2, slot).wait()

        eb = ebuf[eslot].astype(jnp.bfloat16)
        bias_t = bias_ref[:, pl.ds(v * tv, tv)]
        B_p = x_ref.shape[0]
        for i in range(0, B_p, mc):
            acc = lax.dot_general(
                xb_ref[i:i + mc, :], eb,
                dimension_numbers=(((1,), (1,)), ((), ())),
                preferred_element_type=jnp.float32)
            obuf[slot, i:i + mc, :] = acc + bias_t

        ocopy(v, slot).start()

        @pl.when(v == nv - 1)
        def _():
            ocopy(v, slot).wait()
        if nv >= 2:
            @pl.when(v == nv - 1)
            def _():
                ocopy(v - 1, nxt).wait()

    return _logits_kernel


def kernel(x, emb_weight, bias):
    B, D = x.shape
    V = emb_weight.shape[0]

    tv = next((t for t in (1280, 640, 512, 768, 384, 256, 128) if V % t == 0),
              512)
    V_pad = _round_up(V, tv)
    nv = V_pad // tv

    B_p = _round_up(B, 8)
    x_p = x if B_p == B else jnp.pad(x, ((0, B_p - B), (0, 0)))
    bias_p = bias.astype(jnp.float32)
    emb_p = emb_weight
    if V_pad != V:
        bias_p = jnp.pad(bias_p, ((0, 0), (0, V_pad - V)))
        emb_p = jnp.pad(emb_p, ((0, V_pad - V), (0, 0)))

    mc = min(1024, B_p)
    out = pl.pallas_call(
        _make_kernel(tv, nv, mc),
        out_shape=jax.ShapeDtypeStruct((B_p, V_pad), jnp.float32),
        grid=(nv,),
        in_specs=[
            pl.BlockSpec((B_p, D), lambda v: (0, 0)),
            pl.BlockSpec((1, V_pad), lambda v: (0, 0)),
            pl.BlockSpec(memory_space=pl.ANY),
        ],
        out_specs=pl.BlockSpec(memory_space=pl.ANY),
        scratch_shapes=[
            pltpu.VMEM((B_p, D), jnp.bfloat16),
            pltpu.VMEM((3, tv, D), jnp.float32),
            pltpu.VMEM((2, B_p, tv), jnp.float32),
            pltpu.SemaphoreType.DMA((3,)),
            pltpu.SemaphoreType.DMA((2,)),
        ],
        compiler_params=pltpu.CompilerParams(
            dimension_semantics=("arbitrary",),
            vmem_limit_bytes=64 * 1024 * 1024,
        ),
    )(x_p, bias_p, emb_p)
    if B_p != B or V_pad != V:
        out = out[:B, :V]
    return out


# speedup vs baseline: 1.0082x; 1.0029x over previous
import jax
import jax.numpy as jnp
from jax import lax
from jax.experimental import pallas as pl
from jax.experimental.pallas import tpu as pltpu


def _round_up(x, m):
    return (x + m - 1) // m * m


def _make_kernel(tv, nv, mc):
    def _logits_kernel(x_ref, bias_ref, emb_hbm, out_hbm,
                       xb_ref, ebuf, obuf, in_sem, out_sem):
        v = pl.program_id(0)
        slot = lax.rem(v, 2)
        nxt = 1 - slot

        def ecopy(i, s):
            return pltpu.make_async_copy(
                emb_hbm.at[pl.ds(i * tv, tv), :], ebuf.at[s], in_sem.at[s])

        def ocopy(i, s):
            return pltpu.make_async_copy(
                obuf.at[s], out_hbm.at[:, pl.ds(i * tv, tv)], out_sem.at[s])

        @pl.when(v == 0)
        def _():
            ecopy(0, 0).start()
            xb_ref[...] = x_ref[...].astype(jnp.bfloat16)

        @pl.when(v + 1 < nv)
        def _():
            ecopy(v + 1, nxt).start()

        ecopy(v, slot).wait()

        @pl.when(v >= 2)
        def _():
            ocopy(v - 2, slot).wait()

        eb = ebuf[slot].astype(jnp.bfloat16)
        bias_t = bias_ref[:, pl.ds(v * tv, tv)]
        B_p = x_ref.shape[0]
        for i in range(0, B_p, mc):
            acc = lax.dot_general(
                xb_ref[i:i + mc, :], eb,
                dimension_numbers=(((1,), (1,)), ((), ())),
                preferred_element_type=jnp.float32)
            obuf[slot, i:i + mc, :] = acc + bias_t

        ocopy(v, slot).start()

        @pl.when(v == nv - 1)
        def _():
            ocopy(v, slot).wait()
        if nv >= 2:
            @pl.when(v == nv - 1)
            def _():
                ocopy(v - 1, nxt).wait()

    return _logits_kernel


def kernel(x, emb_weight, bias):
    B, D = x.shape
    V = emb_weight.shape[0]

    tv = next((t for t in (1280, 640, 512, 768, 384, 256, 128) if V % t == 0),
              512)
    V_pad = _round_up(V, tv)
    nv = V_pad // tv

    B_p = _round_up(B, 8)
    x_p = x if B_p == B else jnp.pad(x, ((0, B_p - B), (0, 0)))
    bias_p = bias.astype(jnp.float32)
    emb_p = emb_weight
    if V_pad != V:
        bias_p = jnp.pad(bias_p, ((0, 0), (0, V_pad - V)))
        emb_p = jnp.pad(emb_p, ((0, V_pad - V), (0, 0)))

    mc = min(1024, B_p)
    out = pl.pallas_call(
        _make_kernel(tv, nv, mc),
        out_shape=jax.ShapeDtypeStruct((B_p, V_pad), jnp.float32),
        grid=(nv,),
        in_specs=[
            pl.BlockSpec((B_p, D), lambda v: (0, 0)),
            pl.BlockSpec((1, V_pad), lambda v: (0, 0)),
            pl.BlockSpec(memory_space=pl.ANY),
        ],
        out_specs=pl.BlockSpec(memory_space=pl.ANY),
        scratch_shapes=[
            pltpu.VMEM((B_p, D), jnp.bfloat16),
            pltpu.VMEM((2, tv, D), jnp.float32),
            pltpu.VMEM((2, B_p, tv), jnp.float32),
            pltpu.SemaphoreType.DMA((2,)),
            pltpu.SemaphoreType.DMA((2,)),
        ],
        compiler_params=pltpu.CompilerParams(
            dimension_semantics=("arbitrary",),
            vmem_limit_bytes=64 * 1024 * 1024,
        ),
    )(x_p, bias_p, emb_p)
    if B_p != B or V_pad != V:
        out = out[:B, :V]
    return out


# speedup vs baseline: 1.0146x; 1.0064x over previous
import jax
import jax.numpy as jnp
from jax import lax
from jax.experimental import pallas as pl
from jax.experimental.pallas import tpu as pltpu


def _round_up(x, m):
    return (x + m - 1) // m * m


def _make_kernel(tv, nv, mc):
    def _logits_kernel(x_ref, bias_ref, emb_hbm, out_hbm,
                       xb_ref, ebuf, obuf, in_sem, out_sem):
        v = pl.program_id(0)
        slot = lax.rem(v, 2)
        nxt = 1 - slot

        def ecopy(i, s):
            return pltpu.make_async_copy(
                emb_hbm.at[pl.ds(i * tv, tv), :], ebuf.at[s], in_sem.at[s])

        def ocopy(i, s):
            return pltpu.make_async_copy(
                obuf.at[s], out_hbm.at[:, pl.ds(i * tv, tv)], out_sem.at[s])

        @pl.when(v == 0)
        def _():
            ecopy(0, 0).start()
            xb_ref[...] = x_ref[...].astype(jnp.bfloat16)

        @pl.when(v + 1 < nv)
        def _():
            ecopy(v + 1, nxt).start()

        ecopy(v, slot).wait()

        @pl.when(v >= 2)
        def _():
            ocopy(v - 2, slot).wait()

        eb = ebuf[slot].astype(jnp.bfloat16)
        bias_t = bias_ref[:, pl.ds(v * tv, tv)]
        B_p = x_ref.shape[0]
        for i in range(0, B_p, mc):
            m = min(mc, B_p - i)
            acc = lax.dot_general(
                xb_ref[i:i + m, :], eb,
                dimension_numbers=(((1,), (1,)), ((), ())),
                preferred_element_type=jnp.float32)
            obuf[slot, i:i + m, :] = acc + bias_t

        ocopy(v, slot).start()

        @pl.when(v == nv - 1)
        def _():
            ocopy(v, slot).wait()
        if nv >= 2:
            @pl.when(v == nv - 1)
            def _():
                ocopy(v - 1, nxt).wait()

    return _logits_kernel


def kernel(x, emb_weight, bias):
    B, D = x.shape
    V = emb_weight.shape[0]

    tv = next((t for t in (1280, 640, 512, 768, 384, 256, 128) if V % t == 0),
              512)
    V_pad = _round_up(V, tv)
    nv = V_pad // tv

    B_p = _round_up(B, 8)
    x_p = x if B_p == B else jnp.pad(x, ((0, B_p - B), (0, 0)))
    bias_p = bias.astype(jnp.float32)
    emb_p = emb_weight
    if V_pad != V:
        bias_p = jnp.pad(bias_p, ((0, 0), (0, V_pad - V)))
        emb_p = jnp.pad(emb_p, ((0, V_pad - V), (0, 0)))

    mc = min(1024, B_p)
    out = pl.pallas_call(
        _make_kernel(tv, nv, mc),
        out_shape=jax.ShapeDtypeStruct((B_p, V_pad), jnp.float32),
        grid=(nv,),
        in_specs=[
            pl.BlockSpec((B_p, D), lambda v: (0, 0)),
            pl.BlockSpec((1, V_pad), lambda v: (0, 0)),
            pl.BlockSpec(memory_space=pl.ANY),
        ],
        out_specs=pl.BlockSpec(memory_space=pl.ANY),
        scratch_shapes=[
            pltpu.VMEM((B_p, D), jnp.bfloat16),
            pltpu.VMEM((2, tv, D), jnp.float32),
            pltpu.VMEM((2, B_p, tv), jnp.float32),
            pltpu.SemaphoreType.DMA((2,)),
            pltpu.SemaphoreType.DMA((2,)),
        ],
        compiler_params=pltpu.CompilerParams(
            dimension_semantics=("arbitrary",),
            vmem_limit_bytes=64 * 1024 * 1024,
        ),
    )(x_p, bias_p, emb_p)
    if B_p != B or V_pad != V:
        out = out[:B, :V]
    return out
